# Initial kernel scaffold; baseline (speedup 1.0000x reference)
import jax
import jax.numpy as jnp
from jax.experimental import pallas as pl


def kernel(queries, keys, vals, fast_vals, timestamps, topk):
    raise NotImplementedError("write your pallas kernel here")


# speedup vs baseline: 3.4570x; 3.4570x over previous
import functools

import jax
import jax.numpy as jnp
from jax.experimental import pallas as pl
from jax.experimental.pallas import tpu as pltpu

N_SLOTS = 100000
KEY_DIM = 64
Q = 1024
TOPK = 8
CHUNK = 512
NSTEP = 196
SP = CHUNK * NSTEP
NSEG8 = SP // 8
NSEG8_REAL = N_SLOTS // 8
NSEG64 = SP // 64
NEG = -1e30
BIGI = 1 << 30


def _seg8max(x):
    r = x.shape[0]
    return jnp.max(x.reshape(r // 8, 8, x.shape[1]), axis=1)


def _stage1_kernel(qT_ref, keys_ref, m8_ref, cand_ref, m64_scr):
    step = pl.program_id(0)
    scores = jnp.dot(keys_ref[...], qT_ref[...],
                     preferred_element_type=jnp.float32) * 0.125
    m8c = _seg8max(scores)
    srow = (jax.lax.broadcasted_iota(jnp.int32, (CHUNK // 8, Q), 0)
            + step * (CHUNK // 8))
    m8c = jnp.where(srow >= NSEG8_REAL, NEG, m8c)
    m8_ref[...] = m8c
    m64_scr[pl.ds(step * (CHUNK // 64), CHUNK // 64), :] = _seg8max(m8c)

    @pl.when(step == NSTEP - 1)
    def _extract():
        work = m64_scr[...]
        iota = jax.lax.broadcasted_iota(jnp.int32, (NSEG64, Q), 0)
        io8 = jax.lax.broadcasted_iota(jnp.int32, (8, Q), 0)
        for j in range(TOPK):
            m = jnp.max(work, axis=0, keepdims=True)
            am = jnp.min(jnp.where(work == m, iota, BIGI),
                         axis=0, keepdims=True)
            cand_ref[pl.ds(j * 8, 8), :] = am * 8 + io8
            if j < TOPK - 1:
                work = jnp.where(iota == am, NEG, work)


RB = 8192


def _stage2_kernel(kc_ref, qT_ref, out_ref):
    sc = jnp.dot(kc_ref[...], qT_ref[...],
                 preferred_element_type=jnp.float32) * 0.125
    g = jax.lax.broadcasted_iota(jnp.int32, (RB, 128), 0)
    lane = jax.lax.broadcasted_iota(jnp.int32, (RB, 128), 1)
    col = jnp.sum(jnp.where(lane == (g // 64) % 128, sc, 0.0),
                  axis=1, keepdims=True)
    out_ref[...] = jnp.broadcast_to(col, (RB, 8))


def _stage2(kcand, qT):
    return pl.pallas_call(
        _stage2_kernel,
        grid=(Q * 64 // RB,),
        in_specs=[
            pl.BlockSpec((RB, KEY_DIM), lambda i: (i, 0)),
            pl.BlockSpec((KEY_DIM, 128), lambda i: (0, i)),
        ],
        out_specs=pl.BlockSpec((RB, 8), lambda i: (i, 0)),
        out_shape=jax.ShapeDtypeStruct((Q * 64, 8), jnp.float32),
    )(kcand, qT)


@functools.partial(jax.jit, static_argnames=())
def _stage1(qT, keys_pad):
    return pl.pallas_call(
        _stage1_kernel,
        grid=(NSTEP,),
        in_specs=[
            pl.BlockSpec((KEY_DIM, Q), lambda i: (0, 0)),
            pl.BlockSpec((CHUNK, KEY_DIM), lambda i: (i, 0)),
        ],
        out_specs=[
            pl.BlockSpec((CHUNK // 8, Q), lambda i: (i, 0)),
            pl.BlockSpec((64, Q), lambda i: (0, 0)),
        ],
        out_shape=[
            jax.ShapeDtypeStruct((NSEG8, Q), jnp.float32),
            jax.ShapeDtypeStruct((64, Q), jnp.int32),
        ],
        scratch_shapes=[pltpu.VMEM((NSEG64, Q), jnp.float32)],
    )(qT, keys_pad)


def kernel(queries, keys, vals, fast_vals, timestamps, topk):
    del topk
    keys_pad = jnp.pad(keys, ((0, SP - N_SLOTS), (0, 0))).astype(jnp.bfloat16)
    qT = queries.T.astype(jnp.bfloat16)
    m8, cand = _stage1(qT, keys_pad)

    cand_vals = jnp.take_along_axis(m8, cand, axis=0)
    work = cand_vals
    segs8 = []
    for _ in range(TOPK):
        m = jnp.max(work, axis=0, keepdims=True)
        am = jnp.min(jnp.where(work == m, cand, BIGI), axis=0, keepdims=True)
        segs8.append(am)
        work = jnp.where(cand == am, NEG, work)
    segs8 = jnp.concatenate(segs8, axis=0)
    cand_slot = (segs8.T[:, :, None] * 8
                 + jnp.arange(8, dtype=jnp.int32)[None, None, :]
                 ).reshape(Q, 64)
    kcand = jnp.take(keys_pad, cand_slot.reshape(-1), axis=0)
    cs = _stage2(kcand, qT)[:, 0].reshape(Q, 64)
    work = cs
    idxs, vals8 = [], []
    for _ in range(TOPK):
        m = jnp.max(work, axis=-1, keepdims=True)
        am = jnp.min(jnp.where(work == m, cand_slot, BIGI), axis=-1, keepdims=True)
        idxs.append(am)
        vals8.append(m)
        work = jnp.where(cand_slot == am, NEG, work)
    idx = jnp.concatenate(idxs, axis=-1)
    sim_topk = jnp.concatenate(vals8, axis=-1)
    attn = jax.nn.softmax(sim_topk, axis=-1)
    eff = vals + fast_vals
    gathered = jnp.take(eff, idx, axis=0)
    weighted = gathered * attn[..., None]
    ts = jnp.take(timestamps, idx, axis=0)
    return (weighted, idx.astype(jnp.int64), ts)


# speedup vs baseline: 3.5708x; 1.0329x over previous
import functools

import jax
import jax.numpy as jnp
from jax.experimental import pallas as pl
from jax.experimental.pallas import tpu as pltpu

N_SLOTS = 100000
KEY_DIM = 64
Q = 1024
TOPK = 8
CHUNK = 512
NSTEP = 196
SP = CHUNK * NSTEP
NSEG8 = SP // 8
NSEG8_REAL = N_SLOTS // 8
NSEG64 = SP // 64
NEG = -1e30
BIGI = 1 << 30


def _stage1_kernel(qT_ref, keys_ref, m8_ref, cand_ref, m64_scr):
    step = pl.program_id(0)
    scores = jnp.dot(keys_ref[...], qT_ref[...],
                     preferred_element_type=jnp.float32) * 0.125
    m8c = jnp.max(scores.reshape(8, CHUNK // 8, Q), axis=0)
    r = jax.lax.broadcasted_iota(jnp.int32, (CHUNK // 8, Q), 0)
    seg_nat = step * (CHUNK // 8) + (r % 8) * 8 + r // 8
    m8c = jnp.where(seg_nat >= NSEG8_REAL, NEG, m8c)
    m8_ref[...] = m8c
    m64_scr[pl.ds(step * (CHUNK // 64), CHUNK // 64), :] = jnp.max(
        m8c.reshape(8, CHUNK // 64, Q), axis=0)

    @pl.when(step == NSTEP - 1)
    def _extract():
        work = m64_scr[...]
        iota = jax.lax.broadcasted_iota(jnp.int32, (NSEG64, Q), 0)
        io8 = jax.lax.broadcasted_iota(jnp.int32, (8, Q), 0)
        for j in range(TOPK):
            m = jnp.max(work, axis=0, keepdims=True)
            am = jnp.min(jnp.where(work == m, iota, BIGI),
                         axis=0, keepdims=True)
            cand_ref[pl.ds(j * 8, 8), :] = am * 8 + io8
            if j < TOPK - 1:
                work = jnp.where(iota == am, NEG, work)


RB = 8192


def _stage2_kernel(kc_ref, qT_ref, out_ref):
    sc = jnp.dot(kc_ref[...], qT_ref[...],
                 preferred_element_type=jnp.float32) * 0.125
    g = jax.lax.broadcasted_iota(jnp.int32, (RB, 128), 0)
    lane = jax.lax.broadcasted_iota(jnp.int32, (RB, 128), 1)
    col = jnp.sum(jnp.where(lane == (g // 64) % 128, sc, 0.0),
                  axis=1, keepdims=True)
    out_ref[...] = jnp.broadcast_to(col, (RB, 8))


def _stage2(kcand, qT):
    return pl.pallas_call(
        _stage2_kernel,
        grid=(Q * 64 // RB,),
        in_specs=[
            pl.BlockSpec((RB, KEY_DIM), lambda i: (i, 0)),
            pl.BlockSpec((KEY_DIM, 128), lambda i: (0, i)),
        ],
        out_specs=pl.BlockSpec((RB, 8), lambda i: (i, 0)),
        out_shape=jax.ShapeDtypeStruct((Q * 64, 8), jnp.float32),
    )(kcand, qT)


@functools.partial(jax.jit, static_argnames=())
def _stage1(qT, keys_pad):
    return pl.pallas_call(
        _stage1_kernel,
        grid=(NSTEP,),
        in_specs=[
            pl.BlockSpec((KEY_DIM, Q), lambda i: (0, 0)),
            pl.BlockSpec((CHUNK, KEY_DIM), lambda i: (i, 0)),
        ],
        out_specs=[
            pl.BlockSpec((CHUNK // 8, Q), lambda i: (i, 0)),
            pl.BlockSpec((64, Q), lambda i: (0, 0)),
        ],
        out_shape=[
            jax.ShapeDtypeStruct((NSEG8, Q), jnp.float32),
            jax.ShapeDtypeStruct((64, Q), jnp.int32),
        ],
        scratch_shapes=[pltpu.VMEM((NSEG64, Q), jnp.float32)],
    )(qT, keys_pad)


def kernel(queries, keys, vals, fast_vals, timestamps, topk):
    del topk
    keys_pad = jnp.pad(keys, ((0, SP - N_SLOTS), (0, 0))).astype(jnp.bfloat16)
    keys_perm = (keys_pad.reshape(NSTEP, 8, 8, 8, KEY_DIM)
                 .transpose(0, 3, 2, 1, 4).reshape(SP, KEY_DIM))
    qT = queries.T.astype(jnp.bfloat16)
    m8, cand = _stage1(qT, keys_perm)

    cand_row = (cand // 64) * 64 + (cand % 8) * 8 + (cand // 8) % 8
    cand_vals = jnp.take_along_axis(m8, cand_row, axis=0)
    work = cand_vals
    segs8 = []
    for _ in range(TOPK):
        m = jnp.max(work, axis=0, keepdims=True)
        am = jnp.min(jnp.where(work == m, cand, BIGI), axis=0, keepdims=True)
        segs8.append(am)
        work = jnp.where(cand == am, NEG, work)
    segs8 = jnp.concatenate(segs8, axis=0)
    cand_slot = (segs8.T[:, :, None] * 8
                 + jnp.arange(8, dtype=jnp.int32)[None, None, :]
                 ).reshape(Q, 64)
    s = cand_slot.reshape(-1)
    s_perm = ((s // 512) * 512 + (s % 8) * 64
              + ((s % 512) // 8) % 8 * 8 + (s % 512) // 64)
    kcand = jnp.take(keys_perm, s_perm, axis=0)
    cs = _stage2(kcand, qT)[:, 0].reshape(Q, 64)
    work = cs
    idxs, vals8 = [], []
    for _ in range(TOPK):
        m = jnp.max(work, axis=-1, keepdims=True)
        am = jnp.min(jnp.where(work == m, cand_slot, BIGI), axis=-1, keepdims=True)
        idxs.append(am)
        vals8.append(m)
        work = jnp.where(cand_slot == am, NEG, work)
    idx = jnp.concatenate(idxs, axis=-1)
    sim_topk = jnp.concatenate(vals8, axis=-1)
    attn = jax.nn.softmax(sim_topk, axis=-1)
    eff = vals + fast_vals
    gathered = jnp.take(eff, idx, axis=0)
    weighted = gathered * attn[..., None]
    ts = jnp.take(timestamps, idx, axis=0)
    return (weighted, idx.astype(jnp.int64), ts)


# speedup vs baseline: 3.6271x; 1.0158x over previous
import functools

import jax
import jax.numpy as jnp
from jax.experimental import pallas as pl
from jax.experimental.pallas import tpu as pltpu

N_SLOTS = 100000
KEY_DIM = 64
Q = 1024
TOPK = 8
CHUNK = 512
NSTEP = 196
SP = CHUNK * NSTEP
NSEG8 = SP // 8
NSEG8_REAL = N_SLOTS // 8
NSEG64 = SP // 64
NEG = -1e30
BIGI = 1 << 30


def _stage1_kernel(qT_ref, keys_ref, m8_ref, cand_ref, m64_scr):
    step = pl.program_id(0)
    scores = jnp.dot(keys_ref[...], qT_ref[...],
                     preferred_element_type=jnp.float32) * 0.125
    m8c = jnp.max(scores.reshape(8, CHUNK // 8, Q), axis=0)
    r = jax.lax.broadcasted_iota(jnp.int32, (CHUNK // 8, Q), 0)
    seg_nat = step * (CHUNK // 8) + (r % 8) * 8 + r // 8
    m8c = jnp.where(seg_nat >= NSEG8_REAL, NEG, m8c)
    m8_ref[...] = m8c
    m64_scr[pl.ds(step * (CHUNK // 64), CHUNK // 64), :] = jnp.max(
        m8c.reshape(8, CHUNK // 64, Q), axis=0)

    @pl.when(step == NSTEP - 1)
    def _extract():
        work = m64_scr[...]
        iota = jax.lax.broadcasted_iota(jnp.int32, (NSEG64, Q), 0)
        io8 = jax.lax.broadcasted_iota(jnp.int32, (8, Q), 0)
        for j in range(TOPK):
            m = jnp.max(work, axis=0, keepdims=True)
            am = jnp.min(jnp.where(work == m, iota, BIGI),
                         axis=0, keepdims=True)
            cand_ref[pl.ds(j * 8, 8), :] = am * 8 + io8
            if j < TOPK - 1:
                work = jnp.where(iota == am, NEG, work)


def _pick8_kernel(cv_ref, cand_ref, sperm_ref, cslot_ref):
    work = cv_ref[...]
    cand = cand_ref[...]
    io8 = jax.lax.broadcasted_iota(jnp.int32, (8, Q), 0)
    for j in range(TOPK):
        m = jnp.max(work, axis=0, keepdims=True)
        am = jnp.min(jnp.where(work == m, cand, BIGI), axis=0, keepdims=True)
        s = am * 8 + io8
        sp = ((s // 512) * 512 + (s % 8) * 64
              + ((s % 512) // 8) % 8 * 8 + (s % 512) // 64)
        cslot_ref[pl.ds(j * 8, 8), :] = s
        sperm_ref[pl.ds(j * 8, 8), :] = sp
        if j < TOPK - 1:
            work = jnp.where(cand == am, NEG, work)


def _pick8(cand_vals, cand):
    return pl.pallas_call(
        _pick8_kernel,
        out_shape=[
            jax.ShapeDtypeStruct((64, Q), jnp.int32),
            jax.ShapeDtypeStruct((64, Q), jnp.int32),
        ],
    )(cand_vals, cand)


def _final_kernel(cs_ref, cslot_ref, idx_ref, attn_ref):
    work = cs_ref[...]
    cslot = cslot_ref[...]
    vals8, idxs = [], []
    for j in range(TOPK):
        m = jnp.max(work, axis=1, keepdims=True)
        am = jnp.min(jnp.where(work == m, cslot, BIGI), axis=1, keepdims=True)
        vals8.append(m)
        idxs.append(am)
        if j < TOPK - 1:
            work = jnp.where(cslot == am, NEG, work)
    sim8 = jnp.concatenate(vals8, axis=1)
    e = jnp.exp(sim8 - sim8[:, 0:1])
    attn_ref[...] = e / jnp.sum(e, axis=1, keepdims=True)
    idx_ref[...] = jnp.concatenate(idxs, axis=1)


def _final(cs, cslotT):
    return pl.pallas_call(
        _final_kernel,
        out_shape=[
            jax.ShapeDtypeStruct((Q, 8), jnp.int32),
            jax.ShapeDtypeStruct((Q, 8), jnp.float32),
        ],
    )(cs, cslotT)


RB = 8192


def _stage2_kernel(kc_ref, qT_ref, out_ref):
    sc = jnp.dot(kc_ref[...], qT_ref[...],
                 preferred_element_type=jnp.float32) * 0.125
    g = jax.lax.broadcasted_iota(jnp.int32, (RB, 128), 0)
    lane = jax.lax.broadcasted_iota(jnp.int32, (RB, 128), 1)
    col = jnp.sum(jnp.where(lane == (g // 64) % 128, sc, 0.0),
                  axis=1, keepdims=True)
    out_ref[...] = jnp.broadcast_to(col, (RB, 8))


def _stage2(kcand, qT):
    return pl.pallas_call(
        _stage2_kernel,
        grid=(Q * 64 // RB,),
        in_specs=[
            pl.BlockSpec((RB, KEY_DIM), lambda i: (i, 0)),
            pl.BlockSpec((KEY_DIM, 128), lambda i: (0, i)),
        ],
        out_specs=pl.BlockSpec((RB, 8), lambda i: (i, 0)),
        out_shape=jax.ShapeDtypeStruct((Q * 64, 8), jnp.float32),
    )(kcand, qT)


@functools.partial(jax.jit, static_argnames=())
def _stage1(qT, keys_pad):
    return pl.pallas_call(
        _stage1_kernel,
        grid=(NSTEP,),
        in_specs=[
            pl.BlockSpec((KEY_DIM, Q), lambda i: (0, 0)),
            pl.BlockSpec((CHUNK, KEY_DIM), lambda i: (i, 0)),
        ],
        out_specs=[
            pl.BlockSpec((CHUNK // 8, Q), lambda i: (i, 0)),
            pl.BlockSpec((64, Q), lambda i: (0, 0)),
        ],
        out_shape=[
            jax.ShapeDtypeStruct((NSEG8, Q), jnp.float32),
            jax.ShapeDtypeStruct((64, Q), jnp.int32),
        ],
        scratch_shapes=[pltpu.VMEM((NSEG64, Q), jnp.float32)],
    )(qT, keys_pad)


def kernel(queries, keys, vals, fast_vals, timestamps, topk):
    del topk
    keys_pad = jnp.pad(keys, ((0, SP - N_SLOTS), (0, 0))).astype(jnp.bfloat16)
    keys_perm = (keys_pad.reshape(NSTEP, 8, 8, 8, KEY_DIM)
                 .transpose(0, 3, 2, 1, 4).reshape(SP, KEY_DIM))
    qT = queries.T.astype(jnp.bfloat16)
    m8, cand = _stage1(qT, keys_perm)

    cand_row = (cand // 64) * 64 + (cand % 8) * 8 + (cand // 8) % 8
    cand_vals = jnp.take_along_axis(m8, cand_row, axis=0)
    sperm, cslot = _pick8(cand_vals, cand)
    kcand = jnp.take(keys_perm, sperm.T.reshape(-1), axis=0)
    cs = _stage2(kcand, qT)[:, 0].reshape(Q, 64)
    idx, attn = _final(cs, cslot.T)
    gathered = (jnp.take(vals, idx, axis=0)
                + jnp.take(fast_vals, idx, axis=0))
    weighted = gathered * attn[..., None]
    ts = jnp.take(timestamps, idx, axis=0)
    return (weighted, idx.astype(jnp.int64), ts)
